# Initial kernel scaffold; baseline (speedup 1.0000x reference)
import jax
import jax.numpy as jnp
from jax.experimental import pallas as pl


def kernel(fnode_t, fmess_t, agraph_t, bgraph_t, cgraph, scope, fnode_g, fmess_g, agraph_g, bgraph_g, params):
    raise NotImplementedError("write your pallas kernel here")


# speedup vs baseline: 1.9626x; 1.9626x over previous
import functools

import jax
import jax.numpy as jnp
from jax import lax
from jax.experimental import pallas as pl
from jax.experimental.pallas import tpu as pltpu
from jax.experimental.pallas import tpu_sc as plsc

H = 128
_CH = 128
_NW = 32
_GRP = 2


def _sc_gather(table, idx):
    one_d = table.ndim == 1
    D = 1 if one_d else table.shape[1]
    R = idx.shape[0]
    per = _NW * _CH * _GRP
    Rp = ((R + per - 1) // per) * per
    idxp = jnp.pad(idx.astype(jnp.int32), (0, Rp - R))
    nck = Rp // (_NW * _CH)
    grps = nck // _GRP
    idx2 = idxp.reshape(_NW, nck, _CH)
    mesh = plsc.VectorSubcoreMesh(core_axis_name="c", subcore_axis_name="s")
    buf_t = pltpu.VMEM((_CH,) if one_d else (_CH, D), table.dtype)
    out_t = jax.ShapeDtypeStruct((Rp,) if one_d else (Rp, D), table.dtype)

    def body(tab_h, idx_h, out_h, idx_v, b0, b1, s0, s1):
        wid = lax.axis_index("s") * 2 + lax.axis_index("c")
        c0 = wid * nck
        pltpu.sync_copy(idx_h.at[wid], idx_v)

        def grp(g, carry):
            base = (c0 + _GRP * g) * _CH
            cp0 = pltpu.async_copy(tab_h.at[idx_v.at[_GRP * g]], b0, s0)
            cp1 = pltpu.async_copy(tab_h.at[idx_v.at[_GRP * g + 1]], b1, s1)
            cp0.wait()
            pltpu.sync_copy(b0, out_h.at[pl.ds(base, _CH)])
            cp1.wait()
            pltpu.sync_copy(b1, out_h.at[pl.ds(base + _CH, _CH)])
            return carry

        lax.fori_loop(0, grps, grp, 0)

    out = pl.kernel(
        body, out_type=out_t, mesh=mesh,
        scratch_types=[pltpu.VMEM((nck, _CH), jnp.int32), buf_t, buf_t,
                       pltpu.SemaphoreType.DMA, pltpu.SemaphoreType.DMA],
    )(table, idx2)
    return out[:R]


def _sc_gather_small1d(table, idx):
    N = table.shape[0]
    R = idx.shape[0]
    per = _NW * _CH
    Rp = ((R + per - 1) // per) * per
    rpw = Rp // _NW
    idxp = jnp.pad(idx.astype(jnp.int32), (0, Rp - R)).reshape(_NW, rpw)
    mesh = plsc.VectorSubcoreMesh(core_axis_name="c", subcore_axis_name="s")

    def body(tab_h, idx_h, out_h, tab_v, idx_v, out_v):
        wid = lax.axis_index("s") * 2 + lax.axis_index("c")
        pltpu.sync_copy(tab_h, tab_v)
        pltpu.sync_copy(idx_h.at[wid], idx_v)

        def step(j, carry):
            iv = idx_v[pl.ds(j * 16, 16)]
            out_v[pl.ds(j * 16, 16)] = plsc.load_gather(tab_v, [iv])
            return carry

        lax.fori_loop(0, rpw // 16, step, 0)
        pltpu.sync_copy(out_v, out_h.at[pl.ds(wid * rpw, rpw)])

    out = pl.kernel(
        body, out_type=jax.ShapeDtypeStruct((Rp,), table.dtype), mesh=mesh,
        compiler_params=pltpu.CompilerParams(needs_layout_passes=False),
        scratch_types=[pltpu.VMEM((N,), table.dtype),
                       pltpu.VMEM((rpw,), jnp.int32),
                       pltpu.VMEM((rpw,), table.dtype)],
    )(table, idxp)
    return out[:R]


def _dot(a, b):
    return jnp.dot(a, b, preferred_element_type=jnp.float32)


def _row_mask(i, be, h):
    rid = lax.broadcasted_iota(jnp.int32, (be, 1), 0) + i * be
    return h * (rid != 0).astype(h.dtype)


def _ints3(v, be):
    return v.astype(jnp.int32).reshape(v.shape[0] // be, 1, be)


def _onehot(v, be, n):
    return (v[:, None] == lax.broadcasted_iota(jnp.int32, (be, n), 1)
            ).astype(jnp.float32)


def _gru_mats(p, din):
    Wx = jnp.concatenate([p["W_z"][:din], p["W_h"][:din], p["W_r"]], axis=1)
    bzh = jnp.concatenate([p["b_z"], p["b_h"]])[None, :]
    return (Wx, p["W_z"][din:], p["W_h"][din:], p["U_r"], bzh,
            p["b_ur"][None, :])


def _gru_tail(i, be, xw, hn, Wzh, Whh, Ur, bzh, bur, first):
    xz = xw[:, :H] + bzh[:, :H]
    xh = xw[:, H:2 * H] + bzh[:, H:]
    if first:
        h = jax.nn.sigmoid(xz) * jnp.tanh(xh)
    else:
        r1 = xw[:, 2 * H:]
        sum_h = jnp.sum(hn, axis=0)
        u = _dot(hn.reshape(4 * be, H), Ur) + bur
        r = jax.nn.sigmoid(r1[None, :, :] + u.reshape(4, be, H))
        sum_g = jnp.sum(r * hn, axis=0)
        z = jax.nn.sigmoid(xz + _dot(sum_h, Wzh))
        pre = jnp.tanh(xh + _dot(sum_g, Whh))
        h = (1.0 - z) * sum_h + z * pre
    return _row_mask(i, be, h)


def _wspec(arr):
    n = arr.ndim
    return pl.BlockSpec(arr.shape, lambda i, _n=n: (0,) * _n)


def _call(body, nb, be, arrs, specs):
    return pl.pallas_call(
        body,
        grid=(nb,),
        in_specs=specs,
        out_specs=pl.BlockSpec((be, H), lambda i: (i, 0)),
        out_shape=jax.ShapeDtypeStruct((nb * be, H), jnp.float32),
    )(*arrs)


def _gru_graph(a3, b3, p3, hn, mats, E, be, first):
    nb = E // be
    ispec = pl.BlockSpec((1, 1, be), lambda i: (i, 0, 0))
    hnspec = pl.BlockSpec((4, be, H), lambda i: (0, i, 0))

    def body(ar, br, pr, *rest):
        i = pl.program_id(0)
        if first:
            (Wxr, Wzhr, Whhr, Urr, bzhr, burr, out) = rest
            hnv = None
        else:
            (hnr, Wxr, Wzhr, Whhr, Urr, bzhr, burr, out) = rest
            hnv = hnr[...]
        oh = jnp.concatenate(
            [_onehot(ar[0, 0, :], be, 44), _onehot(br[0, 0, :], be, 4),
             _onehot(pr[0, 0, :], be, 20)], axis=1)
        xw = _dot(oh, Wxr[...])
        out[...] = _gru_tail(i, be, xw, hnv, Wzhr[...], Whhr[...], Urr[...],
                             bzhr[...], burr[...], first)

    arrs = [a3, b3, p3] + ([] if first else [hn]) + list(mats)
    specs = [ispec] * 3 + ([] if first else [hnspec]) + [_wspec(m) for m in mats]
    return _call(body, nb, be, arrs, specs)


def _gru_dense(hsrc, p3, hn, mats, E, be, first):
    nb = E // be
    ispec = pl.BlockSpec((1, 1, be), lambda i: (i, 0, 0))
    aspec = pl.BlockSpec((be, H), lambda i: (i, 0))
    hnspec = pl.BlockSpec((4, be, H), lambda i: (0, i, 0))

    def body(hsr, pr, *rest):
        i = pl.program_id(0)
        if first:
            (Wxr, Wzhr, Whhr, Urr, bzhr, burr, out) = rest
            hnv = None
        else:
            (hnr, Wxr, Wzhr, Whhr, Urr, bzhr, burr, out) = rest
            hnv = hnr[...]
        Wx = Wxr[...]
        xw = _dot(hsr[...], Wx[:H]) + _dot(_onehot(pr[0, 0, :], be, 20), Wx[H:])
        out[...] = _gru_tail(i, be, xw, hnv, Wzhr[...], Whhr[...], Urr[...],
                             bzhr[...], burr[...], first)

    arrs = [hsrc, p3] + ([] if first else [hn]) + list(mats)
    specs = [aspec, ispec] + ([] if first else [hnspec]) + [_wspec(m) for m in mats]
    return _call(body, nb, be, arrs, specs)


def _combine_dense(A, hn, W1, W2, b, be, act, mask):
    N = A.shape[0]
    nb = N // be
    K = hn.shape[0]
    b2 = b[None, :]

    def body(ar, hnr, W1r, W2r, br, out):
        i = pl.program_id(0)
        agg = jnp.sum(hnr[...], axis=0)
        h = act(_dot(ar[...], W1r[...]) + _dot(agg, W2r[...]) + br[...])
        out[...] = _row_mask(i, be, h) if mask else h

    specs = [pl.BlockSpec((be, H), lambda i: (i, 0)),
             pl.BlockSpec((K, be, H), lambda i: (0, i, 0)),
             _wspec(W1), _wspec(W2), _wspec(b2)]
    return _call(body, nb, be, [A, hn, W1, W2, b2], specs)


def _combine_onehot(i3, hn, W1, W2, b, be):
    nb = i3.shape[0]
    K = hn.shape[0]
    b2 = b[None, :]

    def body(ir, hnr, W1r, W2r, br, out):
        i = pl.program_id(0)
        agg = jnp.sum(hnr[...], axis=0)
        h = jax.nn.relu(_dot(_onehot(ir[0, 0, :], be, 44), W1r[...])
                        + _dot(agg, W2r[...]) + br[...])
        out[...] = _row_mask(i, be, h)

    specs = [pl.BlockSpec((1, 1, be), lambda i: (i, 0, 0)),
             pl.BlockSpec((K, be, H), lambda i: (0, i, 0)),
             _wspec(W1), _wspec(W2), _wspec(b2)]
    return _call(body, nb, be, [i3, hn, W1, W2, b2], specs)


def _forward(gather, gather1d, fnode_t, fmess_t, agraph_t, bgraph_t, cgraph,
             scope, fnode_g, fmess_g, agraph_g, bgraph_g, params):
    p = params
    EG = fmess_g.shape[0]
    ET = fmess_t.shape[0]
    NA = fnode_g.shape[0]
    NT = fnode_t.shape[0]
    B = scope.shape[0]
    be = 1000
    relu, tanh = jax.nn.relu, jnp.tanh

    src_atom = gather1d(fnode_g.astype(jnp.int32), fmess_g[:, 0])
    a3 = _ints3(src_atom, be)
    b3 = _ints3(fmess_g[:, 2], be)
    pg3 = _ints3(fmess_g[:, 3], be)
    mats_g = _gru_mats(p["gru_graph"], 68)
    h = _gru_graph(a3, b3, pg3, None, mats_g, EG, be, True)
    bgT = jnp.transpose(bgraph_g).reshape(-1)
    for _ in range(2):
        hn = gather(h, bgT).reshape(4, EG, H)
        h = _gru_graph(a3, b3, pg3, hn, mats_g, EG, be, False)
    hn = gather(h, jnp.transpose(agraph_g).reshape(-1)).reshape(4, NA, H)
    Wo, bo = p["Wo_graph"]
    hatom = _combine_onehot(_ints3(fnode_g, be), hn, Wo[:44], Wo[44:], bo, be)

    finput_i = gather(p["E_i"], fnode_t[:, 1])
    hn8 = gather(hatom, jnp.transpose(cgraph).reshape(-1)).reshape(8, NT, H)
    Wi, bi = p["W_i"]
    hnode_i = _combine_dense(finput_i, hn8, Wi[:H], Wi[H:], bi, be, relu, False)
    hsrc_i = gather(hnode_i, fmess_t[:, 0])
    pt3 = _ints3(fmess_t[:, 2], be)
    mats_i = _gru_mats(p["gru_inter"], 148)
    h = _gru_dense(hsrc_i, pt3, None, mats_i, ET, be, True)
    bgtT = jnp.transpose(bgraph_t).reshape(-1)
    for _ in range(2):
        hn = gather(h, bgtT).reshape(4, ET, H)
        h = _gru_dense(hsrc_i, pt3, hn, mats_i, ET, be, False)
    agtT = jnp.transpose(agraph_t).reshape(-1)
    hn = gather(h, agtT).reshape(4, NT, H)
    Woi, boi = p["Wo_inter"]
    hinter = _combine_dense(hnode_i, hn, Woi[:H], Woi[H:], boi, be, relu, True)

    finput_c = gather(p["E_c"], fnode_t[:, 0])
    Wc, bc = p["W_c"]
    hnode_c = _combine_dense(finput_c, hinter[None], Wc[:H], Wc[H:], bc, be,
                             relu, False)
    hsrc_c = gather(hnode_c, fmess_t[:, 0])
    mats_t = _gru_mats(p["gru_tree"], 148)
    h = _gru_dense(hsrc_c, pt3, None, mats_t, ET, be, True)
    for _ in range(2):
        hn = gather(h, bgtT).reshape(4, ET, H)
        h = _gru_dense(hsrc_c, pt3, hn, mats_t, ET, be, False)
    hmess_out = h
    hn = gather(h, agtT).reshape(4, NT, H)
    Wot, bot = p["Wo_tree"]
    hnode_out = _combine_dense(hnode_c, hn, Wot[:H], Wot[H:], bot, be, relu,
                               True)

    roots = scope[:, 0].astype(jnp.int32)
    ag_pad = jnp.pad(agraph_t.astype(jnp.int32), ((0, 0), (0, 124)))
    ag_r = gather(ag_pad, roots)[:, :4]
    hm = gather(hmess_out, jnp.transpose(ag_r).reshape(-1)).reshape(4, B, H)
    fnode_r = gather(hnode_c, roots)
    Wr, br = p["W_root"]
    hroot = _combine_dense(fnode_r, hm, Wr[:H], Wr[H:], br, B, tanh, False)
    return hroot, hnode_out, hinter, hatom


def kernel(fnode_t, fmess_t, agraph_t, bgraph_t, cgraph, scope,
           fnode_g, fmess_g, agraph_g, bgraph_g, params):
    return _forward(_sc_gather, _sc_gather_small1d, fnode_t, fmess_t,
                    agraph_t, bgraph_t, cgraph, scope, fnode_g, fmess_g,
                    agraph_g, bgraph_g, params)
